# Initial kernel scaffold; baseline (speedup 1.0000x reference)
import jax
import jax.numpy as jnp
from jax.experimental import pallas as pl


def kernel(logits, labels, node_embeddings, pos_pairs, neg_pairs, temperature):
    raise NotImplementedError("write your pallas kernel here")


# speedup vs baseline: 1.0187x; 1.0187x over previous
import functools

import jax
import jax.numpy as jnp
from jax import lax
from jax.experimental import pallas as pl
from jax.experimental.pallas import tpu as pltpu
from jax.experimental.pallas import tpu_sc as plsc

_EPS = 1e-8


def _sc_gather(table, idx):
    V, D = table.shape
    B = idx.shape[0]
    info = plsc.get_sparse_core_info()
    NW = info.num_cores * info.num_subcores
    assert B % (8 * NW) == 0
    b_per_w = B // NW
    chunk = min(128, b_per_w)
    assert b_per_w % chunk == 0
    n_chunks = b_per_w // chunk
    mesh = plsc.VectorSubcoreMesh(core_axis_name="c", subcore_axis_name="s")

    @functools.partial(
        pl.kernel,
        mesh=mesh,
        out_type=jax.ShapeDtypeStruct((B, D), jnp.float32),
        scratch_types=[
            pltpu.VMEM((chunk,), jnp.int32),
            pltpu.VMEM((chunk, D), jnp.float32),
            pltpu.SemaphoreType.DMA,
        ],
    )
    def gather_kernel(table_hbm, idx_hbm, out_hbm, idx_v, rows_v, sem):
        wid = lax.axis_index("s") * info.num_cores + lax.axis_index("c")
        base = wid * b_per_w
        for c in range(n_chunks):
            off = base + c * chunk
            pltpu.sync_copy(idx_hbm.at[pl.ds(off, chunk)], idx_v)
            pltpu.async_copy(table_hbm.at[idx_v], rows_v, sem).wait()
            pltpu.sync_copy(rows_v, out_hbm.at[pl.ds(off, chunk)])

    return gather_kernel(table, idx)


def _flash_body(P, NB, KB, n_valid, temp_ref, a_ref, pos_ref, neg_ref,
                lg_ref, lb_ref, out_ref, an_ref, ps_ref, acc_ref):
    k = pl.program_id(0)
    inv_t = 1.0 / temp_ref[0]

    @pl.when(k == 0)
    def _init():
        a = a_ref[...]
        a_n = a / jnp.maximum(
            jnp.sqrt(jnp.sum(a * a, axis=1, keepdims=True)), _EPS)
        an_ref[...] = a_n
        p = pos_ref[...]
        p_n = p / jnp.maximum(
            jnp.sqrt(jnp.sum(p * p, axis=1, keepdims=True)), _EPS)
        ps = jnp.sum(a_n * p_n, axis=1, keepdims=True) * inv_t
        ps_ref[...] = ps
        acc_ref[...] = jnp.exp(ps - inv_t)

    nb = neg_ref[...]
    n_n = nb / jnp.maximum(
        jnp.sqrt(jnp.sum(nb * nb, axis=1, keepdims=True)), _EPS)
    sims = lax.dot_general(
        an_ref[...], n_n, (((1,), (1,)), ((), ())),
        preferred_element_type=jnp.float32,
        precision=lax.Precision.HIGHEST)
    acc_ref[...] += jnp.sum(jnp.exp(sims * inv_t - inv_t), axis=1,
                            keepdims=True)

    @pl.when(k == KB - 1)
    def _finish():
        per_anchor = jnp.log(acc_ref[...]) + inv_t - ps_ref[...]
        nce = jnp.sum(per_anchor) / P
        lg = lg_ref[...]
        lb = lb_ref[...]
        sp = jnp.maximum(-lg, 0.0) + jnp.log1p(jnp.exp(-jnp.abs(lg)))
        bce = jnp.sum(sp + (1.0 - lb) * lg) / n_valid
        out_ref[0, 0] = 0.5 * bce + nce


def _flash_loss(temperature, gathered, logits_pad, labels_pad, P, K, F,
                n_valid):
    NB = 512
    assert K % NB == 0
    KB = K // NB
    rows_l, lanes = logits_pad.shape
    body = functools.partial(_flash_body, P, NB, KB, n_valid)
    out = pl.pallas_call(
        body,
        grid=(KB,),
        in_specs=[
            pl.BlockSpec(memory_space=pltpu.SMEM),
            pl.BlockSpec((P, F), lambda k: (0, 0)),
            pl.BlockSpec((P, F), lambda k: (1, 0)),
            pl.BlockSpec((NB, F), lambda k: (2 * P // NB + k, 0)),
            pl.BlockSpec((rows_l, lanes), lambda k: (0, 0)),
            pl.BlockSpec((rows_l, lanes), lambda k: (0, 0)),
        ],
        out_specs=pl.BlockSpec(memory_space=pltpu.SMEM),
        out_shape=jax.ShapeDtypeStruct((1, 1), jnp.float32),
        scratch_shapes=[
            pltpu.VMEM((P, F), jnp.float32),
            pltpu.VMEM((P, 1), jnp.float32),
            pltpu.VMEM((P, 1), jnp.float32),
        ],
    )(jnp.reshape(temperature, (1,)), gathered, gathered, gathered,
      logits_pad, labels_pad)
    return out[0, 0]


def kernel(logits, labels, node_embeddings, pos_pairs, neg_pairs, temperature):
    N, F = node_embeddings.shape
    P = pos_pairs.shape[1]
    K = neg_pairs.shape[1]

    idx = jnp.concatenate(
        [pos_pairs[0], pos_pairs[1], neg_pairs[1]]).astype(jnp.int32)
    gathered = _sc_gather(node_embeddings, idx)

    lg = jnp.reshape(jnp.squeeze(logits), (-1,))
    n_valid = lg.shape[0]
    n_pad = -n_valid % 1024
    lg_pad = jnp.pad(lg, (0, n_pad), constant_values=40.0)
    lb_pad = jnp.pad(jnp.reshape(labels, (-1,)), (0, n_pad),
                     constant_values=1.0)
    lg_pad = jnp.reshape(lg_pad, (-1, 128))
    lb_pad = jnp.reshape(lb_pad, (-1, 128))

    return _flash_loss(temperature.astype(jnp.float32), gathered,
                       lg_pad, lb_pad, P, K, F, n_valid)


# speedup vs baseline: 1.9400x; 1.9044x over previous
import functools

import jax
import jax.numpy as jnp
from jax import lax
from jax.experimental import pallas as pl
from jax.experimental.pallas import tpu as pltpu
from jax.experimental.pallas import tpu_sc as plsc

_EPS = 1e-8


def _sc_gather(table, idx):
    V, D = table.shape
    B = idx.shape[0]
    info = plsc.get_sparse_core_info()
    NW = info.num_cores * info.num_subcores
    assert B % (8 * NW) == 0
    b_per_w = B // NW
    chunk = min(128, b_per_w)
    assert b_per_w % chunk == 0
    n_chunks = b_per_w // chunk
    mesh = plsc.VectorSubcoreMesh(core_axis_name="c", subcore_axis_name="s")

    @functools.partial(
        pl.kernel,
        mesh=mesh,
        out_type=jax.ShapeDtypeStruct((B, D), jnp.float32),
        scratch_types=[
            pltpu.VMEM((chunk,), jnp.int32),
            pltpu.VMEM((chunk, D), jnp.float32),
            pltpu.SemaphoreType.DMA,
        ],
    )
    def gather_kernel(table_hbm, idx_hbm, out_hbm, idx_v, rows_v, sem):
        wid = lax.axis_index("s") * info.num_cores + lax.axis_index("c")
        base = wid * b_per_w
        for c in range(n_chunks):
            off = base + c * chunk
            pltpu.sync_copy(idx_hbm.at[pl.ds(off, chunk)], idx_v)
            pltpu.async_copy(table_hbm.at[idx_v], rows_v, sem).wait()
            pltpu.sync_copy(rows_v, out_hbm.at[pl.ds(off, chunk)])

    return gather_kernel(table, idx)


def _flash_body(P, NB, KB, n_valid, temp_ref, a_ref, pos_ref, neg_ref,
                lg_ref, lb_ref, out_ref, an_ref, ps_ref, acc_ref):
    k = pl.program_id(0)
    inv_t = 1.0 / temp_ref[0]

    @pl.when(k == 0)
    def _init():
        a = a_ref[...]
        a_n = a / jnp.maximum(
            jnp.sqrt(jnp.sum(a * a, axis=1, keepdims=True)), _EPS)
        an_ref[...] = a_n
        p = pos_ref[...]
        p_n = p / jnp.maximum(
            jnp.sqrt(jnp.sum(p * p, axis=1, keepdims=True)), _EPS)
        ps = jnp.sum(a_n * p_n, axis=1, keepdims=True) * inv_t
        ps_ref[...] = ps
        acc_ref[...] = jnp.exp(ps - inv_t)

    nb = neg_ref[...]
    n_n = nb / jnp.maximum(
        jnp.sqrt(jnp.sum(nb * nb, axis=1, keepdims=True)), _EPS)
    sims = lax.dot_general(
        an_ref[...], n_n, (((1,), (1,)), ((), ())),
        preferred_element_type=jnp.float32,
        precision=lax.Precision.DEFAULT)
    acc_ref[...] += jnp.sum(jnp.exp(sims * inv_t - inv_t), axis=1,
                            keepdims=True)

    @pl.when(k == KB - 1)
    def _finish():
        per_anchor = jnp.log(acc_ref[...]) + inv_t - ps_ref[...]
        nce = jnp.sum(per_anchor) / P
        lg = lg_ref[...]
        lb = lb_ref[...]
        sp = jnp.maximum(-lg, 0.0) + jnp.log1p(jnp.exp(-jnp.abs(lg)))
        bce = jnp.sum(sp + (1.0 - lb) * lg) / n_valid
        out_ref[0, 0] = 0.5 * bce + nce


def _flash_loss(temperature, gathered, logits_pad, labels_pad, P, K, F,
                n_valid):
    NB = 512
    assert K % NB == 0
    KB = K // NB
    rows_l, lanes = logits_pad.shape
    body = functools.partial(_flash_body, P, NB, KB, n_valid)
    out = pl.pallas_call(
        body,
        grid=(KB,),
        in_specs=[
            pl.BlockSpec(memory_space=pltpu.SMEM),
            pl.BlockSpec((P, F), lambda k: (0, 0)),
            pl.BlockSpec((P, F), lambda k: (1, 0)),
            pl.BlockSpec((NB, F), lambda k: (2 * P // NB + k, 0)),
            pl.BlockSpec((rows_l, lanes), lambda k: (0, 0)),
            pl.BlockSpec((rows_l, lanes), lambda k: (0, 0)),
        ],
        out_specs=pl.BlockSpec(memory_space=pltpu.SMEM),
        out_shape=jax.ShapeDtypeStruct((1, 1), jnp.float32),
        scratch_shapes=[
            pltpu.VMEM((P, F), jnp.float32),
            pltpu.VMEM((P, 1), jnp.float32),
            pltpu.VMEM((P, 1), jnp.float32),
        ],
    )(jnp.reshape(temperature, (1,)), gathered, gathered, gathered,
      logits_pad, labels_pad)
    return out[0, 0]


def kernel(logits, labels, node_embeddings, pos_pairs, neg_pairs, temperature):
    N, F = node_embeddings.shape
    P = pos_pairs.shape[1]
    K = neg_pairs.shape[1]

    idx = jnp.concatenate(
        [pos_pairs[0], pos_pairs[1], neg_pairs[1]]).astype(jnp.int32)
    gathered = _sc_gather(node_embeddings, idx)

    lg = jnp.reshape(jnp.squeeze(logits), (-1,))
    n_valid = lg.shape[0]
    n_pad = -n_valid % 1024
    lg_pad = jnp.pad(lg, (0, n_pad), constant_values=40.0)
    lb_pad = jnp.pad(jnp.reshape(labels, (-1,)), (0, n_pad),
                     constant_values=1.0)
    lg_pad = jnp.reshape(lg_pad, (-1, 128))
    lb_pad = jnp.reshape(lb_pad, (-1, 128))

    return _flash_loss(temperature.astype(jnp.float32), gathered,
                       lg_pad, lb_pad, P, K, F, n_valid)
